# baseline (speedup 1.0000x reference)
import functools

import jax
import jax.numpy as jnp
from jax import lax
from jax.experimental import pallas as pl
from jax.experimental.pallas import tpu as pltpu
from jax.experimental.pallas import tpu_sc as plsc

_NUM_FIELDS = 26
_VOCAB = 100000
_HIDDEN = 32
_L = 16
_NC = 2
_NS = 16
_BC = 4096
_U = 16
_VA = 50048
_VB = 49920
_VT = _VA + _VB


@functools.cache
def _build(batch):
  nbc = batch // _BC

  mesh = plsc.VectorSubcoreMesh(
      core_axis_name="c", subcore_axis_name="s",
      num_cores=_NC, num_subcores=_NS)

  @functools.partial(
      pl.kernel,
      out_type=jax.ShapeDtypeStruct((_HIDDEN, batch), jnp.float32),
      mesh=mesh,
      compiler_params=pltpu.CompilerParams(
          use_tc_tiling_on_sc=True, needs_layout_passes=False),
      scratch_types=[
          pltpu.VMEM((_VT + 128,), jnp.float32),
          pltpu.VMEM((_BC,), jnp.int32),
          pltpu.VMEM((_BC,), jnp.int32),
          pltpu.VMEM((batch,), jnp.float32),
          pltpu.SemaphoreType.DMA,
          pltpu.SemaphoreType.DMA,
          pltpu.SemaphoreType.DMA,
          pltpu.SemaphoreType.DMA,
          pltpu.SemaphoreType.DMA,
      ],
  )
  def enc(tt_hbm, tl_hbm, xt_hbm, out_hbm, rowbuf, xc0, xc1, outcol, sx0, sx1,
          sra, srb, stl):
    c = lax.axis_index("c")
    s = lax.axis_index("s")
    h = s * _NC + c

    def xcopy(f, cidx):
      buf, sem = (xc0, sx0) if cidx % 2 == 0 else (xc1, sx1)
      return pltpu.make_async_copy(
          xt_hbm.at[f, pl.ds(cidx * _BC, _BC)], buf, sem)

    def row_a(f):
      return pltpu.make_async_copy(
          tt_hbm.at[f, h, pl.ds(0, _VA)], rowbuf.at[pl.ds(0, _VA)], sra)

    def row_b(f):
      return pltpu.make_async_copy(
          tt_hbm.at[f, h, pl.ds(_VA, _VB)], rowbuf.at[pl.ds(_VA, _VB)], srb)

    def row_tail(f):
      return pltpu.make_async_copy(
          tl_hbm.at[f, h, :], rowbuf.at[pl.ds(_VT, 128)], stl)

    def epass(f, first, half):
      xcopy(f, 0).start()
      for cidx in range(nbc):
        xcopy(f, cidx).wait()
        if cidx + 1 < nbc:
          xcopy(f, cidx + 1).start()

        xbuf = xc0 if cidx % 2 == 0 else xc1

        def body(k, carry):
          for j in range(_U):
            o = (k * _U + j) * _L
            v = xbuf[pl.ds(o, _L)]
            g = plsc.load_gather(rowbuf, [v])
            keep = (v < _VA) if half == 0 else (v >= _VA)
            contrib = jnp.where(keep, g, 0.0)
            if first and half == 0:
              outcol[pl.ds(cidx * _BC + o, _L)] = contrib
            else:
              plsc.addupdate(outcol.at[pl.ds(cidx * _BC + o, _L)], contrib)
          return carry

        lax.fori_loop(0, _BC // (_L * _U), body, 0)

    def field(f, first):
      row_a(f).wait()
      epass(f, first, 0)
      row_b(f).wait()
      row_tail(f).wait()

      @pl.when(f < _NUM_FIELDS - 1)
      def _():
        row_a(f + 1).start()

      epass(f, first, 1)

      @pl.when(f < _NUM_FIELDS - 1)
      def _():
        row_b(f + 1).start()
        row_tail(f + 1).start()

    row_a(0).start()
    row_b(0).start()
    row_tail(0).start()
    field(0, True)

    def fbody(f, carry):
      field(f, False)
      return carry

    lax.fori_loop(1, _NUM_FIELDS, fbody, 0)

    pltpu.sync_copy(outcol, out_hbm.at[h, :])

  return enc


@jax.jit
def kernel(x, tables):
  tt = jnp.transpose(tables, (0, 2, 1))
  tl = jnp.pad(jnp.transpose(tables[:, _VT:, :], (0, 2, 1)),
               ((0, 0), (0, 0), (0, 96)))
  xt = x.astype(jnp.int32).T
  out_t = _build(x.shape[0])(tt, tl, xt)
  return out_t.T


# speedup vs baseline: 1.2348x; 1.2348x over previous
import functools

import jax
import jax.numpy as jnp
from jax import lax
from jax.experimental import pallas as pl
from jax.experimental.pallas import tpu as pltpu
from jax.experimental.pallas import tpu_sc as plsc

_NUM_FIELDS = 26
_VOCAB = 100000
_HIDDEN = 32
_L = 16
_NC = 2
_NS = 16
_BC = 4096
_U = 16
_VA = 50048
_VB = 49920
_VT = _VA + _VB


@functools.cache
def _build(batch):
  nbc = batch // _BC

  mesh = plsc.VectorSubcoreMesh(
      core_axis_name="c", subcore_axis_name="s",
      num_cores=_NC, num_subcores=_NS)

  @functools.partial(
      pl.kernel,
      out_type=jax.ShapeDtypeStruct((_HIDDEN, batch), jnp.float32),
      mesh=mesh,
      compiler_params=pltpu.CompilerParams(
          use_tc_tiling_on_sc=True, needs_layout_passes=False),
      scratch_types=[
          pltpu.VMEM((_VT + 128,), jnp.float32),
          pltpu.VMEM((_BC,), jnp.int32),
          pltpu.VMEM((_BC,), jnp.int32),
          pltpu.VMEM((batch,), jnp.float32),
          pltpu.SemaphoreType.DMA,
          pltpu.SemaphoreType.DMA,
          pltpu.SemaphoreType.DMA,
          pltpu.SemaphoreType.DMA,
          pltpu.SemaphoreType.DMA,
      ],
  )
  def enc(tt_hbm, tl_hbm, xt_hbm, out_hbm, rowbuf, xc0, xc1, outcol, sx0, sx1,
          sra, srb, stl):
    c = lax.axis_index("c")
    s = lax.axis_index("s")
    h = s * _NC + c

    def xcopy(f, cidx):
      buf, sem = (xc0, sx0) if cidx % 2 == 0 else (xc1, sx1)
      return pltpu.make_async_copy(
          xt_hbm.at[f, pl.ds(cidx * _BC, _BC)], buf, sem)

    def row_a(f):
      return pltpu.make_async_copy(
          tt_hbm.at[f, h, pl.ds(0, _VA)], rowbuf.at[pl.ds(0, _VA)], sra)

    def row_b(f):
      return pltpu.make_async_copy(
          tt_hbm.at[f, h, pl.ds(_VA, _VB)], rowbuf.at[pl.ds(_VA, _VB)], srb)

    def row_tail(f):
      return pltpu.make_async_copy(
          tl_hbm.at[f, h, :], rowbuf.at[pl.ds(_VT, 128)], stl)

    def epass(f, first):
      xcopy(f, 0).start()
      for cidx in range(nbc):
        xcopy(f, cidx).wait()
        if cidx + 1 < nbc:
          xcopy(f, cidx + 1).start()

        xbuf = xc0 if cidx % 2 == 0 else xc1

        def body(k, carry):
          for j in range(_U):
            o = (k * _U + j) * _L
            v = xbuf[pl.ds(o, _L)]
            g = plsc.load_gather(rowbuf, [v])
            if first:
              outcol[pl.ds(cidx * _BC + o, _L)] = g
            else:
              plsc.addupdate(outcol.at[pl.ds(cidx * _BC + o, _L)], g)
          return carry

        lax.fori_loop(0, _BC // (_L * _U), body, 0)

    def field(f, first):
      row_a(f).wait()
      row_b(f).wait()
      row_tail(f).wait()

      epass(f, first)

      @pl.when(f < _NUM_FIELDS - 1)
      def _():
        row_a(f + 1).start()
        row_b(f + 1).start()
        row_tail(f + 1).start()

    row_a(0).start()
    row_b(0).start()
    row_tail(0).start()
    field(0, True)

    def fbody(f, carry):
      field(f, False)
      return carry

    lax.fori_loop(1, _NUM_FIELDS, fbody, 0)

    pltpu.sync_copy(outcol, out_hbm.at[h, :])

  return enc


@jax.jit
def kernel(x, tables):
  tt = jnp.transpose(tables, (0, 2, 1))
  tl = jnp.pad(jnp.transpose(tables[:, _VT:, :], (0, 2, 1)),
               ((0, 0), (0, 0), (0, 96)))
  xt = x.astype(jnp.int32).T
  out_t = _build(x.shape[0])(tt, tl, xt)
  return out_t.T


# speedup vs baseline: 1.3096x; 1.0606x over previous
import functools

import jax
import jax.numpy as jnp
from jax import lax
from jax.experimental import pallas as pl
from jax.experimental.pallas import tpu as pltpu
from jax.experimental.pallas import tpu_sc as plsc

_NUM_FIELDS = 26
_VOCAB = 100000
_HIDDEN = 32
_L = 16
_NC = 2
_NS = 16
_BC = 4096
_U = 8


@functools.cache
def _build(batch):
  nbc = batch // _BC

  mesh = plsc.VectorSubcoreMesh(
      core_axis_name="c", subcore_axis_name="s",
      num_cores=_NC, num_subcores=_NS)

  @functools.partial(
      pl.kernel,
      out_type=jax.ShapeDtypeStruct((_HIDDEN, batch), jnp.float32),
      mesh=mesh,
      compiler_params=pltpu.CompilerParams(
          use_tc_tiling_on_sc=True, needs_layout_passes=False),
      scratch_types=[
          pltpu.VMEM((_VOCAB,), jnp.float32),
          pltpu.VMEM((_BC,), jnp.int32),
          pltpu.VMEM((_BC,), jnp.int32),
          pltpu.VMEM((batch,), jnp.float32),
          pltpu.SemaphoreType.DMA,
          pltpu.SemaphoreType.DMA,
      ],
  )
  def enc(tt_hbm, xt_hbm, out_hbm, rowbuf, xc0, xc1, outcol, sx0, sx1):
    c = lax.axis_index("c")
    s = lax.axis_index("s")
    h = s * _NC + c

    def xcopy(f, cidx):
      buf, sem = (xc0, sx0) if cidx % 2 == 0 else (xc1, sx1)
      return pltpu.make_async_copy(
          xt_hbm.at[f, pl.ds(cidx * _BC, _BC)], buf, sem)

    def field(f, first):
      xcopy(f, 0).start()
      pltpu.sync_copy(tt_hbm.at[f, h, :], rowbuf)
      for cidx in range(nbc):
        xcopy(f, cidx).wait()
        if cidx + 1 < nbc:
          xcopy(f, cidx + 1).start()

        xbuf = xc0 if cidx % 2 == 0 else xc1

        def body(k, carry):
          for j in range(_U):
            o = (k * _U + j) * _L
            v = xbuf[pl.ds(o, _L)]
            g = plsc.load_gather(rowbuf, [v])
            if first:
              outcol[pl.ds(cidx * _BC + o, _L)] = g
            else:
              plsc.addupdate(outcol.at[pl.ds(cidx * _BC + o, _L)], g)
          return carry

        lax.fori_loop(0, _BC // (_L * _U), body, 0)

    field(0, True)

    def fbody(f, carry):
      field(f, False)
      return carry

    lax.fori_loop(1, _NUM_FIELDS, fbody, 0)

    pltpu.sync_copy(outcol, out_hbm.at[h, :])

  return enc


@jax.jit
def kernel(x, tables):
  tt = jnp.transpose(tables, (0, 2, 1))
  xt = x.astype(jnp.int32).T
  out_t = _build(x.shape[0])(tt, xt)
  return out_t.T


# speedup vs baseline: 1.5830x; 1.2087x over previous
import functools

import jax
import jax.numpy as jnp
from jax import lax
from jax.experimental import pallas as pl
from jax.experimental.pallas import tpu as pltpu
from jax.experimental.pallas import tpu_sc as plsc

_NUM_FIELDS = 26
_VOCAB = 100000
_HIDDEN = 32
_L = 16
_NC = 2
_NS = 16
_BC = 4096
_U = 8


@functools.cache
def _build(batch):
  nbc = batch // _BC

  mesh = plsc.VectorSubcoreMesh(
      core_axis_name="c", subcore_axis_name="s",
      num_cores=_NC, num_subcores=_NS)

  @functools.partial(
      pl.kernel,
      out_type=jax.ShapeDtypeStruct((_HIDDEN, batch), jnp.float32),
      mesh=mesh,
      compiler_params=pltpu.CompilerParams(
          use_tc_tiling_on_sc=True, needs_layout_passes=False),
      scratch_types=[
          pltpu.VMEM((_VOCAB,), jnp.float32),
          pltpu.VMEM((_BC,), jnp.int32),
          pltpu.VMEM((_BC,), jnp.int32),
          pltpu.VMEM((batch,), jnp.float32),
          pltpu.SemaphoreType.DMA,
          pltpu.SemaphoreType.DMA,
      ],
  )
  def enc(tt_hbm, xt_hbm, out_hbm, rowbuf, xc0, xc1, outcol, sx0, sx1):
    c = lax.axis_index("c")
    s = lax.axis_index("s")
    h = s * _NC + c

    def xcopy(f, cidx):
      buf, sem = (xc0, sx0) if cidx % 2 == 0 else (xc1, sx1)
      return pltpu.make_async_copy(
          xt_hbm.at[f, pl.ds(cidx * _BC, _BC)], buf, sem)

    def field(f, first):
      xcopy(f, 0).start()
      pltpu.sync_copy(tt_hbm.at[f, h, :], rowbuf)
      for cidx in range(nbc):
        xcopy(f, cidx).wait()
        if cidx + 1 < nbc:
          xcopy(f, cidx + 1).start()

        xbuf = xc0 if cidx % 2 == 0 else xc1

        @plsc.parallel_loop(0, _BC // _L, unroll=_U)
        def _(k):
          o = k * _L
          v = xbuf[pl.ds(o, _L)]
          g = plsc.load_gather(rowbuf, [v])
          if first:
            outcol[pl.ds(cidx * _BC + o, _L)] = g
          else:
            plsc.addupdate(outcol.at[pl.ds(cidx * _BC + o, _L)], g)

    field(0, True)

    def fbody(f, carry):
      field(f, False)
      return carry

    lax.fori_loop(1, _NUM_FIELDS, fbody, 0)

    pltpu.sync_copy(outcol, out_hbm.at[h, :])

  return enc


@jax.jit
def kernel(x, tables):
  tt = jnp.transpose(tables, (0, 2, 1))
  xt = x.astype(jnp.int32).T
  out_t = _build(x.shape[0])(tt, xt)
  return out_t.T
